# Initial kernel scaffold; baseline (speedup 1.0000x reference)
import jax
import jax.numpy as jnp
from jax.experimental import pallas as pl


def kernel(x, edge_index, W_l1, W_r1, b1, W_l2, W_r2, b2):
    raise NotImplementedError("write your pallas kernel here")


# speedup vs baseline: 8.6752x; 8.6752x over previous
import functools

import jax
import jax.numpy as jnp
from jax import lax
from jax.experimental import pallas as pl
from jax.experimental.pallas import tpu as pltpu
from jax.experimental.pallas import tpu_sc as plsc

N = 100000
E = 1600000
D_IN = 32
D_HID = 32
D_OUT = 16

NC = 2
NS = 16

CH = 128
NCHUNK = 12544
EPAD = NCHUNK * CH
NPAD = 100096
RPT = NPAD // NS

L1_PER_TILE = NCHUNK // NS
L1_KB = 16
L1_NBLK = L1_PER_TILE // L1_KB
L2_PER_TILE = NCHUNK // (NC * NS)
L2_KB = 8
L2_NBLK = L2_PER_TILE // L2_KB

_mesh = plsc.VectorSubcoreMesh(
    core_axis_name="c", subcore_axis_name="s", num_cores=NC, num_subcores=NS)

_sc_params = pltpu.CompilerParams(use_tc_tiling_on_sc=False)


@functools.partial(
    pl.kernel,
    out_type=(
        jax.ShapeDtypeStruct((NPAD, 16), jnp.float32),
        jax.ShapeDtypeStruct((NPAD, 16), jnp.float32),
        jax.ShapeDtypeStruct((NPAD,), jnp.float32),
    ),
    mesh=_mesh,
    scratch_types=(
        pltpu.VMEM_SHARED((NPAD, 16), jnp.float32),
        pltpu.VMEM_SHARED((NPAD,), jnp.float32),
        pltpu.VMEM((L1_KB, CH), jnp.int32),
        pltpu.VMEM((L1_KB, CH), jnp.int32),
        pltpu.VMEM((CH, 16), jnp.float32),
        pltpu.VMEM((CH,), jnp.float32),
        pltpu.VMEM((RPT,), jnp.float32),
        pltpu.SemaphoreType.DMA,
    ),
    compiler_params=_sc_params,
)
def _sc_layer1(xA, xB, srcc, dstc, zrows, zdeg,
               outA, outB, outD,
               acc, degs, sblk, dblk, rows, ones, degv, sem):
    c = lax.axis_index("c")
    s = lax.axis_index("s")
    r0 = s * RPT

    pltpu.sync_copy(zrows, acc.at[pl.ds(r0, RPT)])

    @pl.when(c == 0)
    def _():
        pltpu.sync_copy(zdeg, degv)
        pltpu.sync_copy(degv, degs.at[pl.ds(r0, RPT)])

    for i in range(CH // 16):
        ones[pl.ds(i * 16, 16)] = jnp.ones((16,), jnp.float32)

    plsc.subcore_barrier()

    t0 = s * L1_PER_TILE

    def edge_loop(table, with_deg):
        def blk_body(b, carry):
            row0 = t0 + b * L1_KB
            pltpu.sync_copy(srcc.at[pl.ds(row0, L1_KB)], sblk)
            pltpu.sync_copy(dstc.at[pl.ds(row0, L1_KB)], dblk)

            def ch_body(j, carry2):
                pltpu.async_copy(table.at[sblk.at[j]], rows, sem).wait()
                pltpu.sync_copy(rows, acc.at[dblk.at[j]], add=True)
                if with_deg:
                    pltpu.sync_copy(ones, degs.at[dblk.at[j]], add=True)
                return carry2

            return lax.fori_loop(0, L1_KB, ch_body, carry)

        lax.fori_loop(0, L1_NBLK, blk_body, 0)

    @pl.when(c == 0)
    def _():
        edge_loop(xA, True)

    @pl.when(c == 1)
    def _():
        edge_loop(xB, False)

    plsc.subcore_barrier()

    @pl.when(c == 0)
    def _():
        pltpu.sync_copy(acc.at[pl.ds(r0, RPT)], outA.at[pl.ds(r0, RPT)])
        pltpu.sync_copy(degs.at[pl.ds(r0, RPT)], degv)
        pltpu.sync_copy(degv, outD.at[pl.ds(r0, RPT)])

    @pl.when(c == 1)
    def _():
        pltpu.sync_copy(acc.at[pl.ds(r0, RPT)], outB.at[pl.ds(r0, RPT)])


@functools.partial(
    pl.kernel,
    out_type=(
        jax.ShapeDtypeStruct((NPAD, 16), jnp.float32),
        jax.ShapeDtypeStruct((NPAD, 16), jnp.float32),
    ),
    mesh=_mesh,
    scratch_types=(
        pltpu.VMEM_SHARED((NPAD, 16), jnp.float32),
        pltpu.VMEM((L2_KB, CH), jnp.int32),
        pltpu.VMEM((L2_KB, CH), jnp.int32),
        pltpu.VMEM((CH, 16), jnp.float32),
        pltpu.SemaphoreType.DMA,
    ),
    compiler_params=_sc_params,
)
def _sc_layer2(z2, srcc, dstc, zrows,
               out0, out1,
               acc, sblk, dblk, rows, sem):
    c = lax.axis_index("c")
    s = lax.axis_index("s")
    r0 = s * RPT

    pltpu.sync_copy(zrows, acc.at[pl.ds(r0, RPT)])
    plsc.subcore_barrier()

    t0 = c * (NCHUNK // NC) + s * L2_PER_TILE

    def blk_body(b, carry):
        row0 = t0 + b * L2_KB
        pltpu.sync_copy(srcc.at[pl.ds(row0, L2_KB)], sblk)
        pltpu.sync_copy(dstc.at[pl.ds(row0, L2_KB)], dblk)

        def ch_body(j, carry2):
            pltpu.async_copy(z2.at[sblk.at[j]], rows, sem).wait()
            pltpu.sync_copy(rows, acc.at[dblk.at[j]], add=True)
            return carry2

        return lax.fori_loop(0, L2_KB, ch_body, carry)

    lax.fori_loop(0, L2_NBLK, blk_body, 0)

    plsc.subcore_barrier()

    @pl.when(c == 0)
    def _():
        pltpu.sync_copy(acc.at[pl.ds(r0, RPT)], out0.at[pl.ds(r0, RPT)])

    @pl.when(c == 1)
    def _():
        pltpu.sync_copy(acc.at[pl.ds(r0, RPT)], out1.at[pl.ds(r0, RPT)])


TCR = 2000
TCG = N // TCR


def _tc1_body(x_ref, sA_ref, sB_ref, dg_ref,
              wl1a_ref, wl1b_ref, wr1_ref, b1_ref, wl2_ref, wr2_ref, b2_ref,
              z2_ref, hb_ref):
    recip = 1.0 / jnp.maximum(dg_ref[...], 1.0)
    h = ((sA_ref[...] * recip) @ wl1a_ref[...]
         + (sB_ref[...] * recip) @ wl1b_ref[...]
         + x_ref[...] @ wr1_ref[...]
         + b1_ref[...])
    h = jnp.maximum(h, 0.0)
    z2_ref[...] = h @ wl2_ref[...]
    hb_ref[...] = h @ wr2_ref[...] + b2_ref[...]


def _tc2_body(s0_ref, s1_ref, dg_ref, hb_ref, out_ref):
    recip = 1.0 / jnp.maximum(dg_ref[...], 1.0)
    out_ref[...] = (s0_ref[...] + s1_ref[...]) * recip + hb_ref[...]


def _row_spec(cols):
    return pl.BlockSpec((TCR, cols), lambda i: (i, 0))


def _full_spec(r, cols):
    return pl.BlockSpec((r, cols), lambda i: (0, 0))


def kernel(x, edge_index, W_l1, W_r1, b1, W_l2, W_r2, b2):
    src = edge_index[0]
    dst = edge_index[1]

    pad = EPAD - E
    ar = jnp.arange(pad, dtype=jnp.int32)
    src_p = jnp.concatenate([src, (ar * 97) % N]).reshape(NCHUNK, CH)
    dst_p = jnp.concatenate([dst, N + (ar % 8)]).reshape(NCHUNK, CH)

    xA = x[:, :16]
    xB = x[:, 16:]
    zrows = jnp.zeros((RPT, 16), jnp.float32)
    zdeg = jnp.zeros((RPT,), jnp.float32)

    summedA, summedB, deg = _sc_layer1(xA, xB, src_p, dst_p, zrows, zdeg)
    deg2 = deg[:N].reshape(N, 1)

    z2, hb = pl.pallas_call(
        _tc1_body,
        grid=(TCG,),
        in_specs=[
            _row_spec(D_IN), _row_spec(16), _row_spec(16), _row_spec(1),
            _full_spec(16, D_HID), _full_spec(16, D_HID),
            _full_spec(D_IN, D_HID), _full_spec(1, D_HID),
            _full_spec(D_HID, D_OUT), _full_spec(D_HID, D_OUT),
            _full_spec(1, D_OUT),
        ],
        out_specs=[_row_spec(D_OUT), _row_spec(D_OUT)],
        out_shape=[
            jax.ShapeDtypeStruct((N, D_OUT), jnp.float32),
            jax.ShapeDtypeStruct((N, D_OUT), jnp.float32),
        ],
    )(x, summedA[:N], summedB[:N], deg2,
      W_l1[:16], W_l1[16:], W_r1, b1.reshape(1, -1),
      W_l2, W_r2, b2.reshape(1, -1))

    p0, p1 = _sc_layer2(z2, src_p, dst_p, zrows)

    out = pl.pallas_call(
        _tc2_body,
        grid=(TCG,),
        in_specs=[_row_spec(D_OUT), _row_spec(D_OUT), _row_spec(1),
                  _row_spec(D_OUT)],
        out_specs=_row_spec(D_OUT),
        out_shape=jax.ShapeDtypeStruct((N, D_OUT), jnp.float32),
    )(p0[:N], p1[:N], deg2, hb)

    return out


# speedup vs baseline: 15.7021x; 1.8100x over previous
import functools

import jax
import jax.numpy as jnp
from jax import lax
from jax.experimental import pallas as pl
from jax.experimental.pallas import tpu as pltpu
from jax.experimental.pallas import tpu_sc as plsc

N = 100000
E = 1600000
D_IN = 32
D_HID = 32
D_OUT = 16

NC = 2
NS = 16

CH = 128
NCHUNK = 12544
EPAD = NCHUNK * CH
NPAD = 100096
RPT = NPAD // NS

L1_PER_TILE = NCHUNK // NS
L1_KB = 8
L2_PER_TILE = NCHUNK // (NC * NS)
L2_KB = 8
DZ = RPT // 2

_mesh = plsc.VectorSubcoreMesh(
    core_axis_name="c", subcore_axis_name="s", num_cores=NC, num_subcores=NS)

_sc_params = pltpu.CompilerParams(use_tc_tiling_on_sc=False)


def _edge_pipeline(table, srcc, dstc, acc, t0, nchunks, kb,
                   sblk, dblk, rowsb, gsem, ssem, isem,
                   deg=None, ones=None, dsem=None, zdeg=None):
    gpb = kb // 4
    nblk = nchunks // kb
    ngroups = nchunks // 4

    def drain(dummy_src, dst, sem):
        pltpu.make_async_copy(dummy_src, dst, sem).wait()

    rows_dummy = table.at[pl.ds(0, CH)]

    pltpu.sync_copy(srcc.at[pl.ds(t0, kb)], sblk.at[0])
    pltpu.sync_copy(dstc.at[pl.ds(t0, kb)], dblk.at[0])
    for k in range(4):
        pltpu.async_copy(table.at[sblk.at[0, k]], rowsb.at[k], gsem.at[k])

    def group(g, p, b, rg):
        for k in range(4):
            q = p * 4 + k
            drain(rows_dummy, rowsb.at[q], gsem.at[q])
            pltpu.async_copy(rowsb.at[q], acc.at[dblk.at[b & 1, rg * 4 + k]],
                             ssem.at[q], add=True)
            if deg is not None:
                pltpu.async_copy(ones, deg.at[dblk.at[b & 1, rg * 4 + k]],
                                 dsem.at[q], add=True)

        @pl.when((rg == gpb - 1) & (b + 1 < nblk))
        def _():
            drain(srcc.at[pl.ds(t0, kb)], sblk.at[(b + 1) & 1], isem.at[0])
            drain(srcc.at[pl.ds(t0, kb)], dblk.at[(b + 1) & 1], isem.at[1])

        last = rg == gpb - 1
        for k in range(4):
            q = (1 - p) * 4 + k

            @pl.when(g >= 1)
            def _():
                drain(rows_dummy, rowsb.at[q], ssem.at[q])
                if deg is not None:
                    drain(zdeg.at[pl.ds(0, CH)], ones, dsem.at[q])

            @pl.when(g + 1 < ngroups)
            def _():
                slot2 = jnp.where(last, (b + 1) & 1, b & 1)
                row2 = jnp.where(last, k, (rg + 1) * 4 + k)
                pltpu.async_copy(table.at[sblk.at[slot2, row2]],
                                 rowsb.at[q], gsem.at[q])

        @pl.when((rg == 0) & (b + 1 < nblk))
        def _():
            slot = (b + 1) & 1
            off = t0 + (b + 1) * kb
            pltpu.async_copy(srcc.at[pl.ds(off, kb)], sblk.at[slot],
                             isem.at[0])
            pltpu.async_copy(dstc.at[pl.ds(off, kb)], dblk.at[slot],
                             isem.at[1])

        b2 = b + last
        rg2 = jnp.where(last, 0, rg + 1)
        return b2, rg2

    def body(i, carry):
        b, rg = carry
        g0 = i * 2
        b, rg = group(g0, 0, b, rg)
        b, rg = group(g0 + 1, 1, b, rg)
        return b, rg

    lax.fori_loop(0, ngroups // 2, body, (jnp.int32(0), jnp.int32(0)))

    pf = (ngroups - 1) % 2
    for k in range(4):
        q = pf * 4 + k
        drain(rows_dummy, rowsb.at[q], ssem.at[q])
        if deg is not None:
            drain(zdeg.at[pl.ds(0, CH)], ones, dsem.at[q])


@functools.partial(
    pl.kernel,
    out_type=(
        jax.ShapeDtypeStruct((NPAD, 16), jnp.float32),
        jax.ShapeDtypeStruct((NPAD, 16), jnp.float32),
        jax.ShapeDtypeStruct((NPAD,), jnp.float32),
    ),
    mesh=_mesh,
    scratch_types=(
        pltpu.VMEM_SHARED((NPAD, 16), jnp.float32),
        pltpu.VMEM_SHARED((NPAD,), jnp.float32),
        pltpu.VMEM((2, L1_KB, CH), jnp.int32),
        pltpu.VMEM((2, L1_KB, CH), jnp.int32),
        pltpu.VMEM((8, CH, 16), jnp.float32),
        pltpu.VMEM((CH,), jnp.float32),
        pltpu.VMEM((DZ,), jnp.float32),
        pltpu.SemaphoreType.DMA((8,)),
        pltpu.SemaphoreType.DMA((8,)),
        pltpu.SemaphoreType.DMA((8,)),
        pltpu.SemaphoreType.DMA((2,)),
    ),
    compiler_params=_sc_params,
)
def _sc_layer1(xA, xB, srcc, dstc, zrows, zdeg,
               outA, outB, outD,
               acc, degs, sblk, dblk, rowsb, ones, degv,
               gsem, ssem, dsem, isem):
    c = lax.axis_index("c")
    s = lax.axis_index("s")
    r0 = s * RPT

    pltpu.sync_copy(zrows, acc.at[pl.ds(r0, RPT)])

    @pl.when(c == 0)
    def _():
        pltpu.sync_copy(zdeg, degv)
        for i in range(2):
            pltpu.sync_copy(degv, degs.at[pl.ds(r0 + i * DZ, DZ)])

    for i in range(CH // 16):
        ones[pl.ds(i * 16, 16)] = jnp.ones((16,), jnp.float32)

    plsc.subcore_barrier()

    t0 = s * L1_PER_TILE

    @pl.when(c == 0)
    def _():
        _edge_pipeline(xA, srcc, dstc, acc, t0, L1_PER_TILE, L1_KB,
                       sblk, dblk, rowsb, gsem, ssem, isem,
                       deg=degs, ones=ones, dsem=dsem, zdeg=zdeg)

    @pl.when(c == 1)
    def _():
        _edge_pipeline(xB, srcc, dstc, acc, t0, L1_PER_TILE, L1_KB,
                       sblk, dblk, rowsb, gsem, ssem, isem)

    plsc.subcore_barrier()

    @pl.when(c == 0)
    def _():
        pltpu.sync_copy(acc.at[pl.ds(r0, RPT)], outA.at[pl.ds(r0, RPT)])
        for i in range(2):
            pltpu.sync_copy(degs.at[pl.ds(r0 + i * DZ, DZ)], degv)
            pltpu.sync_copy(degv, outD.at[pl.ds(r0 + i * DZ, DZ)])

    @pl.when(c == 1)
    def _():
        pltpu.sync_copy(acc.at[pl.ds(r0, RPT)], outB.at[pl.ds(r0, RPT)])


@functools.partial(
    pl.kernel,
    out_type=(
        jax.ShapeDtypeStruct((NPAD, 16), jnp.float32),
        jax.ShapeDtypeStruct((NPAD, 16), jnp.float32),
    ),
    mesh=_mesh,
    scratch_types=(
        pltpu.VMEM_SHARED((NPAD, 16), jnp.float32),
        pltpu.VMEM((2, L2_KB, CH), jnp.int32),
        pltpu.VMEM((2, L2_KB, CH), jnp.int32),
        pltpu.VMEM((8, CH, 16), jnp.float32),
        pltpu.SemaphoreType.DMA((8,)),
        pltpu.SemaphoreType.DMA((8,)),
        pltpu.SemaphoreType.DMA((2,)),
    ),
    compiler_params=_sc_params,
)
def _sc_layer2(z2, srcc, dstc, zrows,
               out0, out1,
               acc, sblk, dblk, rowsb, gsem, ssem, isem):
    c = lax.axis_index("c")
    s = lax.axis_index("s")
    r0 = s * RPT

    pltpu.sync_copy(zrows, acc.at[pl.ds(r0, RPT)])
    plsc.subcore_barrier()

    t0 = c * (NCHUNK // NC) + s * L2_PER_TILE
    _edge_pipeline(z2, srcc, dstc, acc, t0, L2_PER_TILE, L2_KB,
                   sblk, dblk, rowsb, gsem, ssem, isem)

    plsc.subcore_barrier()

    @pl.when(c == 0)
    def _():
        pltpu.sync_copy(acc.at[pl.ds(r0, RPT)], out0.at[pl.ds(r0, RPT)])

    @pl.when(c == 1)
    def _():
        pltpu.sync_copy(acc.at[pl.ds(r0, RPT)], out1.at[pl.ds(r0, RPT)])


TCR = 2000
TCG = N // TCR


def _tc1_body(x_ref, sA_ref, sB_ref, dg_ref,
              wl1a_ref, wl1b_ref, wr1_ref, b1_ref, wl2_ref, wr2_ref, b2_ref,
              z2_ref, hb_ref):
    recip = 1.0 / jnp.maximum(dg_ref[...], 1.0)
    h = ((sA_ref[...] * recip) @ wl1a_ref[...]
         + (sB_ref[...] * recip) @ wl1b_ref[...]
         + x_ref[...] @ wr1_ref[...]
         + b1_ref[...])
    h = jnp.maximum(h, 0.0)
    z2_ref[...] = h @ wl2_ref[...]
    hb_ref[...] = h @ wr2_ref[...] + b2_ref[...]


def _tc2_body(s0_ref, s1_ref, dg_ref, hb_ref, out_ref):
    recip = 1.0 / jnp.maximum(dg_ref[...], 1.0)
    out_ref[...] = (s0_ref[...] + s1_ref[...]) * recip + hb_ref[...]


def _row_spec(cols):
    return pl.BlockSpec((TCR, cols), lambda i: (i, 0))


def _full_spec(r, cols):
    return pl.BlockSpec((r, cols), lambda i: (0, 0))


def kernel(x, edge_index, W_l1, W_r1, b1, W_l2, W_r2, b2):
    src = edge_index[0]
    dst = edge_index[1]

    pad = EPAD - E
    ar = jnp.arange(pad, dtype=jnp.int32)
    src_p = jnp.concatenate([src, (ar * 97) % N]).reshape(NCHUNK, CH)
    dst_p = jnp.concatenate([dst, N + (ar % 8)]).reshape(NCHUNK, CH)

    xA = x[:, :16]
    xB = x[:, 16:]
    zrows = jnp.zeros((RPT, 16), jnp.float32)
    zdeg = jnp.zeros((DZ,), jnp.float32)

    summedA, summedB, deg = _sc_layer1(xA, xB, src_p, dst_p, zrows, zdeg)
    deg2 = deg[:N].reshape(N, 1)

    z2, hb = pl.pallas_call(
        _tc1_body,
        grid=(TCG,),
        in_specs=[
            _row_spec(D_IN), _row_spec(16), _row_spec(16), _row_spec(1),
            _full_spec(16, D_HID), _full_spec(16, D_HID),
            _full_spec(D_IN, D_HID), _full_spec(1, D_HID),
            _full_spec(D_HID, D_OUT), _full_spec(D_HID, D_OUT),
            _full_spec(1, D_OUT),
        ],
        out_specs=[_row_spec(D_OUT), _row_spec(D_OUT)],
        out_shape=[
            jax.ShapeDtypeStruct((N, D_OUT), jnp.float32),
            jax.ShapeDtypeStruct((N, D_OUT), jnp.float32),
        ],
    )(x, summedA[:N], summedB[:N], deg2,
      W_l1[:16], W_l1[16:], W_r1, b1.reshape(1, -1),
      W_l2, W_r2, b2.reshape(1, -1))

    p0, p1 = _sc_layer2(z2, src_p, dst_p, zrows)

    out = pl.pallas_call(
        _tc2_body,
        grid=(TCG,),
        in_specs=[_row_spec(D_OUT), _row_spec(D_OUT), _row_spec(1),
                  _row_spec(D_OUT)],
        out_specs=_row_spec(D_OUT),
        out_shape=jax.ShapeDtypeStruct((N, D_OUT), jnp.float32),
    )(p0[:N], p1[:N], deg2, hb)

    return out


# speedup vs baseline: 17.6756x; 1.1257x over previous
import functools

import jax
import jax.numpy as jnp
from jax import lax
from jax.experimental import pallas as pl
from jax.experimental.pallas import tpu as pltpu
from jax.experimental.pallas import tpu_sc as plsc

N = 100000
E = 1600000
D_IN = 32
D_HID = 32
D_OUT = 16

NC = 2
NS = 16

CH = 128
NCHUNK = 12544
EPAD = NCHUNK * CH
NPAD = 100096
RPT = NPAD // NS

L1_PER_TILE = NCHUNK // NS
L1_KB = 8
L2_PER_TILE = NCHUNK // (NC * NS)
L2_KB = 8
DZ = RPT // 2

_mesh = plsc.VectorSubcoreMesh(
    core_axis_name="c", subcore_axis_name="s", num_cores=NC, num_subcores=NS)

_sc_params = pltpu.CompilerParams(use_tc_tiling_on_sc=False)


def _edge_pipeline(table, srcc, dstc, acc, t0, nchunks, kb,
                   sblk, dblk, rowsb, gsem, ssem, isem,
                   deg=None, ones=None, dsem=None, zdeg=None):
    gpb = kb // 4
    nblk = nchunks // kb
    ngroups = nchunks // 4

    def drain(dummy_src, dst, sem):
        pltpu.make_async_copy(dummy_src, dst, sem).wait()

    rows_dummy = table.at[pl.ds(0, CH)]

    pltpu.sync_copy(srcc.at[pl.ds(t0, kb)], sblk.at[0])
    pltpu.sync_copy(dstc.at[pl.ds(t0, kb)], dblk.at[0])
    for k in range(4):
        pltpu.async_copy(table.at[sblk.at[0, k]], rowsb.at[k], gsem.at[k])

    def group(g, p, b, rg):
        for k in range(4):
            q = p * 4 + k
            drain(rows_dummy, rowsb.at[q], gsem.at[q])
            pltpu.async_copy(rowsb.at[q], acc.at[dblk.at[b & 1, rg * 4 + k]],
                             ssem.at[q], add=True)
            if deg is not None:
                pltpu.async_copy(ones, deg.at[dblk.at[b & 1, rg * 4 + k]],
                                 dsem.at[q], add=True)

        @pl.when((rg == gpb - 1) & (b + 1 < nblk))
        def _():
            drain(srcc.at[pl.ds(t0, kb)], sblk.at[(b + 1) & 1], isem.at[0])
            drain(srcc.at[pl.ds(t0, kb)], dblk.at[(b + 1) & 1], isem.at[1])

        last = rg == gpb - 1
        for k in range(4):
            q = (1 - p) * 4 + k

            @pl.when(g >= 1)
            def _():
                drain(rows_dummy, rowsb.at[q], ssem.at[q])
                if deg is not None:
                    drain(zdeg.at[pl.ds(0, CH)], ones, dsem.at[q])

            @pl.when(g + 1 < ngroups)
            def _():
                slot2 = jnp.where(last, (b + 1) & 1, b & 1)
                row2 = jnp.where(last, k, (rg + 1) * 4 + k)
                pltpu.async_copy(table.at[sblk.at[slot2, row2]],
                                 rowsb.at[q], gsem.at[q])

        @pl.when((rg == 0) & (b + 1 < nblk))
        def _():
            slot = (b + 1) & 1
            off = t0 + (b + 1) * kb
            pltpu.async_copy(srcc.at[pl.ds(off, kb)], sblk.at[slot],
                             isem.at[0])
            pltpu.async_copy(dstc.at[pl.ds(off, kb)], dblk.at[slot],
                             isem.at[1])

        b2 = b + last
        rg2 = jnp.where(last, 0, rg + 1)
        return b2, rg2

    def body(i, carry):
        b, rg = carry
        g0 = i * 2
        b, rg = group(g0, 0, b, rg)
        b, rg = group(g0 + 1, 1, b, rg)
        return b, rg

    lax.fori_loop(0, ngroups // 2, body, (jnp.int32(0), jnp.int32(0)))

    pf = (ngroups - 1) % 2
    for k in range(4):
        q = pf * 4 + k
        drain(rows_dummy, rowsb.at[q], ssem.at[q])
        if deg is not None:
            drain(zdeg.at[pl.ds(0, CH)], ones, dsem.at[q])


@functools.partial(
    pl.kernel,
    out_type=(
        jax.ShapeDtypeStruct((NPAD, 16), jnp.float32),
        jax.ShapeDtypeStruct((NPAD, 16), jnp.float32),
        jax.ShapeDtypeStruct((NPAD,), jnp.float32),
    ),
    mesh=_mesh,
    scratch_types=(
        pltpu.VMEM_SHARED((NPAD, 16), jnp.float32),
        pltpu.VMEM_SHARED((NPAD,), jnp.float32),
        pltpu.VMEM((2, L1_KB, CH), jnp.int32),
        pltpu.VMEM((2, L1_KB, CH), jnp.int32),
        pltpu.VMEM((8, CH, 16), jnp.float32),
        pltpu.VMEM((CH,), jnp.float32),
        pltpu.VMEM((DZ,), jnp.float32),
        pltpu.SemaphoreType.DMA((8,)),
        pltpu.SemaphoreType.DMA((8,)),
        pltpu.SemaphoreType.DMA((8,)),
        pltpu.SemaphoreType.DMA((2,)),
    ),
    compiler_params=_sc_params,
)
def _sc_layer1(xA, xB, srcc, dstc, zrows, zdeg,
               outA, outB, outD,
               acc, degs, sblk, dblk, rowsb, ones, degv,
               gsem, ssem, dsem, isem):
    c = lax.axis_index("c")
    s = lax.axis_index("s")
    r0 = s * RPT

    pltpu.sync_copy(zrows, acc.at[pl.ds(r0, RPT)])

    @pl.when(c == 0)
    def _():
        pltpu.sync_copy(zdeg, degv)
        for i in range(2):
            pltpu.sync_copy(degv, degs.at[pl.ds(r0 + i * DZ, DZ)])

    for i in range(CH // 16):
        ones[pl.ds(i * 16, 16)] = jnp.ones((16,), jnp.float32)

    plsc.subcore_barrier()

    t0 = s * L1_PER_TILE

    @pl.when(c == 0)
    def _():
        _edge_pipeline(xA, srcc, dstc, acc, t0, L1_PER_TILE, L1_KB,
                       sblk, dblk, rowsb, gsem, ssem, isem,
                       deg=degs, ones=ones, dsem=dsem, zdeg=zdeg)

    @pl.when(c == 1)
    def _():
        _edge_pipeline(xB, srcc, dstc, acc, t0, L1_PER_TILE, L1_KB,
                       sblk, dblk, rowsb, gsem, ssem, isem)

    plsc.subcore_barrier()

    @pl.when(c == 0)
    def _():
        pltpu.sync_copy(acc.at[pl.ds(r0, RPT)], outA.at[pl.ds(r0, RPT)])
        for i in range(2):
            pltpu.sync_copy(degs.at[pl.ds(r0 + i * DZ, DZ)], degv)
            pltpu.sync_copy(degv, outD.at[pl.ds(r0 + i * DZ, DZ)])

    @pl.when(c == 1)
    def _():
        pltpu.sync_copy(acc.at[pl.ds(r0, RPT)], outB.at[pl.ds(r0, RPT)])


@functools.partial(
    pl.kernel,
    out_type=(
        jax.ShapeDtypeStruct((NPAD, 16), jnp.float32),
        jax.ShapeDtypeStruct((NPAD, 16), jnp.float32),
    ),
    mesh=_mesh,
    scratch_types=(
        pltpu.VMEM_SHARED((NPAD, 16), jnp.float32),
        pltpu.VMEM((2, L2_KB, CH), jnp.int32),
        pltpu.VMEM((2, L2_KB, CH), jnp.int32),
        pltpu.VMEM((8, CH, 16), jnp.float32),
        pltpu.SemaphoreType.DMA((8,)),
        pltpu.SemaphoreType.DMA((8,)),
        pltpu.SemaphoreType.DMA((2,)),
    ),
    compiler_params=_sc_params,
)
def _sc_layer2(z2, srcc, dstc, zrows,
               out0, out1,
               acc, sblk, dblk, rowsb, gsem, ssem, isem):
    c = lax.axis_index("c")
    s = lax.axis_index("s")
    r0 = s * RPT

    pltpu.sync_copy(zrows, acc.at[pl.ds(r0, RPT)])
    plsc.subcore_barrier()

    t0 = c * (NCHUNK // NC) + s * L2_PER_TILE
    _edge_pipeline(z2, srcc, dstc, acc, t0, L2_PER_TILE, L2_KB,
                   sblk, dblk, rowsb, gsem, ssem, isem)

    plsc.subcore_barrier()

    @pl.when(c == 0)
    def _():
        pltpu.sync_copy(acc.at[pl.ds(r0, RPT)], out0.at[pl.ds(r0, RPT)])

    @pl.when(c == 1)
    def _():
        pltpu.sync_copy(acc.at[pl.ds(r0, RPT)], out1.at[pl.ds(r0, RPT)])


TCR = 2000
TCG = N // TCR


def _tc1_body(x_ref, sA_ref, sB_ref, dg_ref,
              wl1a_ref, wl1b_ref, wr1_ref, b1_ref, wl2_ref, wr2_ref, b2_ref,
              z2_ref, hb_ref):
    recip = 1.0 / jnp.maximum(dg_ref[...], 1.0)
    h = ((sA_ref[...] * recip) @ wl1a_ref[...]
         + (sB_ref[...] * recip) @ wl1b_ref[...]
         + x_ref[...] @ wr1_ref[...]
         + b1_ref[...])
    h = jnp.maximum(h, 0.0)
    z2_ref[...] = h @ wl2_ref[...]
    hb_ref[...] = h @ wr2_ref[...] + b2_ref[...]


def _tc2_body(s0_ref, s1_ref, dg_ref, hb_ref, out_ref):
    recip = 1.0 / jnp.maximum(dg_ref[...], 1.0)
    out_ref[...] = (s0_ref[...] + s1_ref[...]) * recip + hb_ref[...]


def _row_spec(cols):
    return pl.BlockSpec((TCR, cols), lambda i: (i, 0))


def _full_spec(r, cols):
    return pl.BlockSpec((r, cols), lambda i: (0, 0))


def kernel(x, edge_index, W_l1, W_r1, b1, W_l2, W_r2, b2):
    src = edge_index[0]
    dst = edge_index[1]

    pad = EPAD - E
    ar = jnp.arange(pad, dtype=jnp.int32)
    src_p = jnp.concatenate([src, (ar * 97) % N]).reshape(NCHUNK, CH)
    dst_p = jnp.concatenate([dst, N + (ar % 8)]).reshape(NCHUNK, CH)

    xA = x[:, :16]
    xB = x[:, 16:]
    zrows = jnp.zeros((RPT, 16), jnp.float32)
    zdeg = jnp.zeros((DZ,), jnp.float32)

    summedA, summedB, deg = _sc_layer1(xA, xB, src_p, dst_p, zrows, zdeg)
    deg2 = deg[:N].reshape(N, 1)

    z2, hb = pl.pallas_call(
        _tc1_body,
        grid=(TCG,),
        in_specs=[
            _row_spec(D_IN), _row_spec(16), _row_spec(16), _row_spec(1),
            _full_spec(16, D_HID), _full_spec(16, D_HID),
            _full_spec(D_IN, D_HID), _full_spec(1, D_HID),
            _full_spec(D_HID, D_OUT), _full_spec(D_HID, D_OUT),
            _full_spec(1, D_OUT),
        ],
        out_specs=[_row_spec(D_OUT), _row_spec(D_OUT)],
        out_shape=[
            jax.ShapeDtypeStruct((N, D_OUT), jnp.float32),
            jax.ShapeDtypeStruct((N, D_OUT), jnp.float32),
        ],
    )(x, summedA, summedB, deg2,
      W_l1[:16], W_l1[16:], W_r1, b1.reshape(1, -1),
      W_l2, W_r2, b2.reshape(1, -1))

    p0, p1 = _sc_layer2(z2, src_p, dst_p, zrows)

    out = pl.pallas_call(
        _tc2_body,
        grid=(TCG,),
        in_specs=[_row_spec(D_OUT), _row_spec(D_OUT), _row_spec(1),
                  _row_spec(D_OUT)],
        out_specs=_row_spec(D_OUT),
        out_shape=jax.ShapeDtypeStruct((N, D_OUT), jnp.float32),
    )(p0, p1, deg2, hb)

    return out
